# baseline (speedup 1.0000x reference)
import functools

import jax
import jax.numpy as jnp
from jax import lax
from jax.experimental import pallas as pl
from jax.experimental.pallas import tpu as pltpu
from jax.experimental.pallas import tpu_sc as plsc

NC = 2
NS = 16
L = 16
CH = 128


def _rsqrt(x):
    i = lax.bitcast_convert_type(x, jnp.int32)
    i = jnp.int32(0x5F3759DF) - lax.shift_right_logical(i, 1)
    y = lax.bitcast_convert_type(i, jnp.float32)
    for _ in range(3):
        y = y * (jnp.float32(1.5) - jnp.float32(0.5) * x * y * y)
    return y


def kernel(triplet_idx, entity_embedding, relation_embedding):
    B = triplet_idx.shape[0]
    D = entity_embedding.shape[1]
    NW = NC * NS
    bpw = B // NW
    nch = bpw // CH
    qv = D // L

    mesh = plsc.VectorSubcoreMesh(
        core_axis_name="c", subcore_axis_name="s", num_cores=NC, num_subcores=NS
    )

    @functools.partial(
        pl.kernel,
        out_type=jax.ShapeDtypeStruct((B,), jnp.float32),
        mesh=mesh,
        compiler_params=pltpu.CompilerParams(
            needs_layout_passes=False, use_tc_tiling_on_sc=False
        ),
        scratch_types=[
            pltpu.VMEM((bpw * 3,), jnp.int32),
            pltpu.VMEM((3, nch, CH), jnp.int32),
            pltpu.VMEM((nch, CH, D), jnp.float32),
            pltpu.VMEM((nch, CH, D), jnp.float32),
            pltpu.VMEM((nch, CH, D), jnp.float32),
            pltpu.VMEM((bpw,), jnp.float32),
            pltpu.SemaphoreType.DMA,
        ],
    )
    def k(tri_hbm, ent_hbm, rel_hbm, out_hbm, tri_v, idx_v, hbuf, rbuf, tbuf,
          outv, sem):
        wid = lax.axis_index("s") * NC + lax.axis_index("c")
        base = wid * bpw

        pltpu.sync_copy(tri_hbm.at[pl.ds(base * 3, bpw * 3)], tri_v)

        iota = lax.iota(jnp.int32, L)
        for g in range(bpw // L):
            ri = (g * L + iota) * 3
            j, o = divmod(g * L, CH)
            for c in range(3):
                col = plsc.load_gather(tri_v, [ri + c])
                idx_v[c, j, pl.ds(o, L)] = col

        copies = []
        for j in range(nch):
            copies.append(pltpu.async_copy(ent_hbm.at[idx_v.at[0, j]], hbuf.at[j], sem))
            copies.append(pltpu.async_copy(rel_hbm.at[idx_v.at[1, j]], rbuf.at[j], sem))
            copies.append(pltpu.async_copy(ent_hbm.at[idx_v.at[2, j]], tbuf.at[j], sem))
        for cdesc in copies:
            cdesc.wait()

        eps = jnp.float32(1e-6)
        tiny = jnp.float32(1e-24)

        for j in range(nch):
            def group(g, _, j=j):
                acc = jnp.zeros((L,), jnp.float32)
                for k in range(L):
                    i = g * L + k
                    h = [hbuf[j, i, pl.ds(q * L, L)] for q in range(qv)]
                    t = [tbuf[j, i, pl.ds(q * L, L)] for q in range(qv)]
                    hh = h[0] * h[0]
                    tt = t[0] * t[0]
                    for q in range(1, qv):
                        hh = hh + h[q] * h[q]
                        tt = tt + t[q] * t[q]
                    ih = _rsqrt(jnp.maximum(jnp.sum(hh), tiny))
                    it = _rsqrt(jnp.maximum(jnp.sum(tt), tiny))
                    r = [rbuf[j, i, pl.ds(q * L, L)] for q in range(qv)]
                    s = jnp.abs(h[0] * ih + (r[0] + eps) - t[0] * it)
                    for q in range(1, qv):
                        s = s + jnp.abs(h[q] * ih + (r[q] + eps) - t[q] * it)
                    acc = jnp.where(iota == k, jnp.sum(s), acc)
                outv[pl.ds(j * CH + g * L, L)] = acc
                return 0
            lax.fori_loop(0, CH // L, group, 0)

        pltpu.sync_copy(outv, out_hbm.at[pl.ds(base, bpw)])

    return k(triplet_idx.reshape(-1), entity_embedding, relation_embedding)


# speedup vs baseline: 1.1388x; 1.1388x over previous
import functools

import jax
import jax.numpy as jnp
from jax import lax
from jax.experimental import pallas as pl
from jax.experimental.pallas import tpu as pltpu
from jax.experimental.pallas import tpu_sc as plsc

NC = 2
NS = 16
L = 16
CH = 128


def _rsqrt(x):
    i = lax.bitcast_convert_type(x, jnp.int32)
    i = jnp.int32(0x5F3759DF) - lax.shift_right_logical(i, 1)
    y = lax.bitcast_convert_type(i, jnp.float32)
    for _ in range(3):
        y = y * (jnp.float32(1.5) - jnp.float32(0.5) * x * y * y)
    return y


def kernel(triplet_idx, entity_embedding, relation_embedding):
    B = triplet_idx.shape[0]
    D = entity_embedding.shape[1]
    NW = NC * NS
    bpw = B // NW
    nch = bpw // CH
    qv = D // L

    mesh = plsc.VectorSubcoreMesh(
        core_axis_name="c", subcore_axis_name="s", num_cores=NC, num_subcores=NS
    )

    @functools.partial(
        pl.kernel,
        out_type=jax.ShapeDtypeStruct((B,), jnp.float32),
        mesh=mesh,
        compiler_params=pltpu.CompilerParams(
            needs_layout_passes=False, use_tc_tiling_on_sc=False
        ),
        scratch_types=[
            pltpu.VMEM((bpw * 3,), jnp.int32),
            pltpu.VMEM((3, nch, CH), jnp.int32),
            pltpu.VMEM((nch, CH, D), jnp.float32),
            pltpu.VMEM((nch, CH, D), jnp.float32),
            pltpu.VMEM((nch, CH, D), jnp.float32),
            pltpu.VMEM((bpw,), jnp.float32),
            pltpu.SemaphoreType.DMA,
        ],
    )
    def k(tri_hbm, ent_hbm, rel_hbm, out_hbm, tri_v, idx_v, hbuf, rbuf, tbuf,
          outv, sem):
        wid = lax.axis_index("s") * NC + lax.axis_index("c")
        base = wid * bpw

        pltpu.sync_copy(tri_hbm.at[pl.ds(base * 3, bpw * 3)], tri_v)

        iota = lax.iota(jnp.int32, L)
        for g in range(bpw // L):
            ri = (g * L + iota) * 3
            j, o = divmod(g * L, CH)
            for c in range(3):
                col = plsc.load_gather(tri_v, [ri + c])
                idx_v[c, j, pl.ds(o, L)] = col

        copies = []
        for j in range(nch):
            copies.append(pltpu.async_copy(ent_hbm.at[idx_v.at[0, j]], hbuf.at[j], sem))
            copies.append(pltpu.async_copy(rel_hbm.at[idx_v.at[1, j]], rbuf.at[j], sem))
            copies.append(pltpu.async_copy(ent_hbm.at[idx_v.at[2, j]], tbuf.at[j], sem))

        eps = jnp.float32(1e-6)
        tiny = jnp.float32(1e-24)

        for j in range(nch):
            for cdesc in copies[3 * j:3 * j + 3]:
                cdesc.wait()

            def group(g, _, j=j):
                hsv = jnp.zeros((L,), jnp.float32)
                tsv = jnp.zeros((L,), jnp.float32)
                for k in range(L):
                    i = g * L + k
                    h = [hbuf[j, i, pl.ds(q * L, L)] for q in range(qv)]
                    t = [tbuf[j, i, pl.ds(q * L, L)] for q in range(qv)]
                    hh = h[0] * h[0]
                    tt = t[0] * t[0]
                    for q in range(1, qv):
                        hh = hh + h[q] * h[q]
                        tt = tt + t[q] * t[q]
                    hsv = jnp.where(iota == k, jnp.sum(hh), hsv)
                    tsv = jnp.where(iota == k, jnp.sum(tt), tsv)
                ihv = _rsqrt(jnp.maximum(hsv, tiny))
                itv = _rsqrt(jnp.maximum(tsv, tiny))
                acc = jnp.zeros((L,), jnp.float32)
                for k in range(L):
                    i = g * L + k
                    ih = ihv[k]
                    it = itv[k]
                    h = [hbuf[j, i, pl.ds(q * L, L)] for q in range(qv)]
                    t = [tbuf[j, i, pl.ds(q * L, L)] for q in range(qv)]
                    r = [rbuf[j, i, pl.ds(q * L, L)] for q in range(qv)]
                    s = jnp.abs(h[0] * ih + (r[0] + eps) - t[0] * it)
                    for q in range(1, qv):
                        s = s + jnp.abs(h[q] * ih + (r[q] + eps) - t[q] * it)
                    acc = jnp.where(iota == k, jnp.sum(s), acc)
                outv[pl.ds(j * CH + g * L, L)] = acc
                return 0
            lax.fori_loop(0, CH // L, group, 0)

        pltpu.sync_copy(outv, out_hbm.at[pl.ds(base, bpw)])

    return k(triplet_idx.reshape(-1), entity_embedding, relation_embedding)
